# baseline (speedup 1.0000x reference)
import functools
import math

import jax
import jax.numpy as jnp
from jax import lax
from jax.experimental import pallas as pl
from jax.experimental.pallas import tpu as pltpu
from jax.experimental.pallas import tpu_sc as plsc

D_MODEL = 64
SCALE = math.sqrt(D_MODEL)
NC = 2
NS = 16
NW = NC * NS
CH = 128


def _make_kernel(steps):
    mesh = plsc.VectorSubcoreMesh(core_axis_name="c", subcore_axis_name="s")
    n_rows = NW * steps * CH

    @functools.partial(
        pl.kernel,
        mesh=mesh,
        out_type=jax.ShapeDtypeStruct((n_rows, D_MODEL), jnp.float32),
        scratch_types=[
            pltpu.VMEM((steps, CH), jnp.int32),
            pltpu.VMEM((CH, D_MODEL), jnp.float32),
            pltpu.SemaphoreType.DMA,
        ],
        compiler_params=pltpu.CompilerParams(use_tc_tiling_on_sc=False),
    )
    def emb_kernel(idx_hbm, table_hbm, out_hbm, idx_v, rows_v, gsem):
        wid = lax.axis_index("s") * NC + lax.axis_index("c")
        pltpu.sync_copy(idx_hbm.at[wid], idx_v)

        def step(j, carry):
            pltpu.async_copy(table_hbm.at[idx_v.at[j]], rows_v, gsem).wait()

            def rowloop(r, c2):
                for c in range(D_MODEL // 16):
                    sl = pl.ds(c * 16, 16)
                    rows_v[r, sl] = rows_v[r, sl] * SCALE
                return c2

            lax.fori_loop(0, CH, rowloop, 0)
            row0 = (wid * steps + j) * CH
            pltpu.sync_copy(rows_v, out_hbm.at[pl.ds(row0, CH)])
            return carry

        lax.fori_loop(0, steps, step, 0)

    return emb_kernel


def kernel(x, table):
    b, s = x.shape
    n = b * s
    assert n % (NW * CH) == 0, (b, s)
    steps = n // (NW * CH)
    idx = x.reshape(NW, steps, CH)
    out = _make_kernel(steps)(idx, table)
    return out.reshape(b, s, D_MODEL)


# speedup vs baseline: 1.1901x; 1.1901x over previous
import functools
import math

import jax
import jax.numpy as jnp
from jax import lax
from jax.experimental import pallas as pl
from jax.experimental.pallas import tpu as pltpu
from jax.experimental.pallas import tpu_sc as plsc

D_MODEL = 64
SCALE = math.sqrt(D_MODEL)
NC = 2
NS = 16
NW = NC * NS
CH = 128
NBUF = 4


def _make_kernel(steps):
    mesh = plsc.VectorSubcoreMesh(core_axis_name="c", subcore_axis_name="s")
    n_rows = NW * steps * CH
    assert steps >= 2 * NBUF and steps % NBUF == 0, steps

    @functools.partial(
        pl.kernel,
        mesh=mesh,
        out_type=jax.ShapeDtypeStruct((n_rows, D_MODEL), jnp.float32),
        scratch_types=[
            pltpu.VMEM((steps, CH), jnp.int32),
            pltpu.VMEM((NBUF, CH, D_MODEL), jnp.float32),
            [pltpu.SemaphoreType.DMA] * NBUF,
            [pltpu.SemaphoreType.DMA] * NBUF,
        ],
        compiler_params=pltpu.CompilerParams(use_tc_tiling_on_sc=False),
    )
    def emb_kernel(idx_hbm, table_hbm, out_hbm, idx_v, rows_v, gs, ss):
        wid = lax.axis_index("s") * NC + lax.axis_index("c")
        pltpu.sync_copy(idx_hbm.at[wid], idx_v)
        out_base = wid * steps

        def gather_start(j, b):
            pltpu.async_copy(table_hbm.at[idx_v.at[j]], rows_v.at[b], gs[b])

        def gather_wait(j, b):
            pltpu.make_async_copy(
                table_hbm.at[idx_v.at[j]], rows_v.at[b], gs[b]
            ).wait()

        def scatter_start(j, b):
            row0 = (out_base + j) * CH
            pltpu.async_copy(rows_v.at[b], out_hbm.at[pl.ds(row0, CH)], ss[b])

        def scatter_wait(j, b):
            row0 = (out_base + j) * CH
            pltpu.make_async_copy(
                rows_v.at[b], out_hbm.at[pl.ds(row0, CH)], ss[b]
            ).wait()

        def scale(b):
            @plsc.parallel_loop(0, CH, 1, unroll=4)
            def _(r):
                for c in range(D_MODEL // 16):
                    sl = pl.ds(c * 16, 16)
                    rows_v[b, r, sl] = rows_v[b, r, sl] * SCALE

        def process(j, b, wait_prev_scatter, prefetch):
            gather_wait(j, b)
            scale(b)
            scatter_start(j, b)
            if prefetch:
                b2 = (b + 2) % NBUF
                if wait_prev_scatter:
                    scatter_wait(j - 2, b2)
                gather_start(j + 2, b2)

        gather_start(0, 0)
        gather_start(1, 1)
        process(0, 0, False, True)
        process(1, 1, False, True)

        @pl.loop(2, steps - 2, step=NBUF)
        def _(j0):
            for i in range(NBUF):
                process(j0 + i, (i + 2) % NBUF, True, True)

        process(steps - 2, (steps - 2) % NBUF, False, False)
        process(steps - 1, (steps - 1) % NBUF, False, False)
        for j in range(steps - 4, steps):
            scatter_wait(j, j % NBUF)

    return emb_kernel


def kernel(x, table):
    b, s = x.shape
    n = b * s
    assert n % (NW * CH) == 0, (b, s)
    steps = n // (NW * CH)
    idx = x.reshape(NW, steps, CH)
    out = _make_kernel(steps)(idx, table)
    return out.reshape(b, s, D_MODEL)
